# baseline (speedup 1.0000x reference)
import functools

import jax
import jax.numpy as jnp
from jax import lax
from jax.experimental import pallas as pl
from jax.experimental.pallas import tpu as pltpu
from jax.experimental.pallas import tpu_sc as plsc

N = 10000
E = 160000
NTILES = 16
NPT = 640
NPAD = NTILES * NPT
CHUNK = 128
GROUP = 8
NGROUP = 10
EPT_PAD = GROUP * NGROUP * CHUNK
E_PAD = EPT_PAD * NTILES


def _sc_aggregate(xa, xb, idxs, zrow):
    mesh = plsc.VectorSubcoreMesh(core_axis_name="c", subcore_axis_name="s")

    @functools.partial(
        pl.kernel,
        out_type=(
            jax.ShapeDtypeStruct((NPAD, 128), jnp.float32),
            jax.ShapeDtypeStruct((NPAD, 128), jnp.float32),
        ),
        mesh=mesh,
        scratch_types=[
            pltpu.VMEM_SHARED((NPAD, 128), jnp.float32),
            pltpu.VMEM((2 * GROUP, CHUNK), jnp.int32),
            pltpu.VMEM((CHUNK, 128), jnp.float32),
            pltpu.VMEM((CHUNK, 128), jnp.float32),
            pltpu.SemaphoreType.DMA,
            pltpu.SemaphoreType.DMA,
            pltpu.SemaphoreType.DMA,
            pltpu.SemaphoreType.DMA,
        ],
    )
    def k(xa_h, xb_h, idxs_h, zrow_h, agga_h, aggb_h,
          agg_s, idx_v, rowbuf0, rowbuf1, gsem0, gsem1, ssem0,
          ssem1):
        c = lax.axis_index("c")
        s = lax.axis_index("s")
        base = s * NPT
        sl = pl.ds(base, NPT)
        pltpu.sync_copy(zrow_h, agg_s.at[sl])
        plsc.subcore_barrier()

        bufs = (rowbuf0, rowbuf1)
        gsems = (gsem0, gsem1)
        ssems = (ssem0, ssem1)

        def run(x_h):
            def group(g, carry):
                pltpu.sync_copy(idxs_h.at[s * NGROUP + g], idx_v)
                gh = [None] * GROUP
                sh = [None] * GROUP
                gh[0] = pltpu.async_copy(x_h.at[idx_v.at[0]], bufs[0],
                                         gsems[0])
                for j in range(GROUP):
                    b = j % 2
                    gh[j].wait()
                    if j + 1 < GROUP:
                        if j >= 1:
                            sh[j - 1].wait()
                        gh[j + 1] = pltpu.async_copy(
                            x_h.at[idx_v.at[j + 1]], bufs[1 - b],
                            gsems[1 - b])
                    sh[j] = pltpu.async_copy(bufs[b],
                                             agg_s.at[idx_v.at[GROUP + j]],
                                             ssems[b], add=True)
                sh[GROUP - 2].wait()
                sh[GROUP - 1].wait()
                return carry
            lax.fori_loop(0, NGROUP, group, 0)

        @pl.when(c == 0)
        def _():
            run(xa_h)

        @pl.when(c == 1)
        def _():
            run(xb_h)

        plsc.subcore_barrier()

        @pl.when(c == 0)
        def _():
            pltpu.sync_copy(agg_s.at[sl], agga_h.at[sl])

        @pl.when(c == 1)
        def _():
            pltpu.sync_copy(agg_s.at[sl], aggb_h.at[sl])

    return k(xa, xb, idxs, zrow)


def _sc_count(idxs, zrow, ones):
    mesh = plsc.VectorSubcoreMesh(core_axis_name="c", subcore_axis_name="s")
    gpw = NTILES * NGROUP // 32

    @functools.partial(
        pl.kernel,
        out_type=(
            jax.ShapeDtypeStruct((NPAD, 128), jnp.float32),
            jax.ShapeDtypeStruct((NPAD, 128), jnp.float32),
        ),
        mesh=mesh,
        scratch_types=[
            pltpu.VMEM_SHARED((NPAD, 128), jnp.float32),
            pltpu.VMEM((2 * GROUP, CHUNK), jnp.int32),
            pltpu.VMEM((CHUNK, 128), jnp.float32),
            pltpu.SemaphoreType.DMA,
        ],
    )
    def k(idxs_h, zrow_h, ones_h, cntl_h, cntr_h, cnt_s, idx_v, ones_v,
          sem):
        c = lax.axis_index("c")
        s = lax.axis_index("s")
        base = s * NPT
        sl = pl.ds(base, NPT)
        gbase = (c * NTILES + s) * gpw
        pltpu.sync_copy(zrow_h, cnt_s.at[sl])
        pltpu.sync_copy(ones_h, ones_v)
        plsc.subcore_barrier()

        def group(g, carry):
            pltpu.sync_copy(idxs_h.at[gbase + g], idx_v)
            hs = [pltpu.async_copy(ones_v, cnt_s.at[idx_v.at[GROUP + j]],
                                   sem, add=True)
                  for j in range(GROUP)]
            for h in hs:
                h.wait()
            return carry
        lax.fori_loop(0, gpw, group, 0)

        plsc.subcore_barrier()

        @pl.when(c == 0)
        def _():
            pltpu.sync_copy(cnt_s.at[sl], cntl_h.at[sl])

        @pl.when(c == 1)
        def _():
            pltpu.sync_copy(cnt_s.at[sl], cntr_h.at[sl])

    return k(idxs, zrow, ones)


def _mlp_factors(a0, a1, a2, a3, cntl, cntr, x, wlt, bl2, wrt, wat, ba2,
                 w1t, b12, w2t, b22, w3t, b32):
    rb = 1024

    def body(r0, r1, r2, r3, cl, cr, xr, wl_r, bl_r, wr_r, wa_r, ba_r,
             w1_r, b1_r, w2_r, b2_r, w3_r, b3_r, p_ref, qt_ref):
        cntc = jnp.maximum(cl[:, :1] + cr[:, :1], 1.0)
        agg = jnp.concatenate([r0[...], r1[...], r2[...], r3[...]],
                              axis=1) / cntc
        h = (jnp.dot(agg, wl_r[...], preferred_element_type=jnp.float32)
             + bl_r[...]
             + jnp.dot(xr[...], wr_r[...],
                       preferred_element_type=jnp.float32))
        h = jnp.maximum(h, 0.0)
        h = jnp.maximum(
            jnp.dot(h, wa_r[...], preferred_element_type=jnp.float32)
            + ba_r[...], 0.0)
        h = jnp.maximum(
            jnp.dot(h, w1_r[...], preferred_element_type=jnp.float32)
            + b1_r[...], 0.0)
        h = jnp.maximum(
            jnp.dot(h, w2_r[...], preferred_element_type=jnp.float32)
            + b2_r[...], 0.0)
        z = (jnp.dot(h, w3_r[...], preferred_element_type=jnp.float32)
             + b3_r[...])
        sq = jnp.sum(z * z, axis=1, keepdims=True)
        col = lax.broadcasted_iota(jnp.int32, z.shape, 1)
        p = jnp.where(col < 3, z, jnp.where(col == 3, sq, 0.0))
        p_ref[...] = p
        qt_ref[...] = p.T

    full = lambda shape: pl.BlockSpec(shape, lambda i: (0, 0))
    return pl.pallas_call(
        body,
        grid=(pl.cdiv(N, rb),),
        in_specs=[
            pl.BlockSpec((rb, 128), lambda i: (i, 0)),
            pl.BlockSpec((rb, 128), lambda i: (i, 0)),
            pl.BlockSpec((rb, 128), lambda i: (i, 0)),
            pl.BlockSpec((rb, 128), lambda i: (i, 0)),
            pl.BlockSpec((rb, 128), lambda i: (i, 0)),
            pl.BlockSpec((rb, 128), lambda i: (i, 0)),
            pl.BlockSpec((rb, 512), lambda i: (i, 0)),
            full((512, 256)), full((1, 256)), full((512, 256)),
            full((256, 128)), full((1, 128)),
            full((128, 64)), full((1, 64)), full((64, 32)), full((1, 32)),
            full((32, 8)), full((1, 8)),
        ],
        out_specs=[
            pl.BlockSpec((rb, 8), lambda i: (i, 0)),
            pl.BlockSpec((8, rb), lambda i: (0, i)),
        ],
        out_shape=[
            jax.ShapeDtypeStruct((N, 8), jnp.float32),
            jax.ShapeDtypeStruct((8, N), jnp.float32),
        ],
    )(a0, a1, a2, a3, cntl, cntr, x, wlt, bl2, wrt, wat, ba2, w1t, b12,
      w2t, b22, w3t, b32)


def _cdist(p, qt):
    rb, cb = 512, 4096

    def body(p_ref, qt_ref, o_ref):
        pm = p_ref[...]
        qm = qt_ref[...]
        colp = lax.broadcasted_iota(jnp.int32, pm.shape, 1)
        colq = lax.broadcasted_iota(jnp.int32, qm.shape, 0)
        zi = jnp.where(colp < 3, pm, 0.0)
        zj = jnp.where(colq < 3, qm, 0.0)
        zz = jnp.dot(zi, zj, preferred_element_type=jnp.float32)
        d2 = (pm[:, 3:4] + qm[3:4, :]) - 2.0 * zz
        o_ref[...] = jnp.sqrt(jnp.maximum(d2, 1e-24))

    return pl.pallas_call(
        body,
        grid=(pl.cdiv(N, rb), pl.cdiv(N, cb)),
        in_specs=[
            pl.BlockSpec((rb, 8), lambda i, j: (i, 0)),
            pl.BlockSpec((8, cb), lambda i, j: (0, j)),
        ],
        out_specs=pl.BlockSpec((rb, cb), lambda i, j: (i, j)),
        out_shape=jax.ShapeDtypeStruct((N, N), jnp.float32),
    )(p, qt)


def kernel(x, edge_index, Wl, bl, Wr, Wa, ba, W1, b1, W2, b2, W3, b3):
    f32 = jnp.float32
    src = edge_index[0].astype(jnp.int32)
    dst = edge_index[1].astype(jnp.int32)
    npad_e = E_PAD - E
    pad_iota = jnp.arange(npad_e, dtype=jnp.int32)
    src_full = jnp.concatenate([src, pad_iota % N])
    dst_full = jnp.concatenate([dst, N + pad_iota % (NPAD - N)])
    srcs = src_full.reshape(NTILES * NGROUP, GROUP, CHUNK)
    dsts = dst_full.reshape(NTILES * NGROUP, GROUP, CHUNK)
    idxs = jnp.concatenate([srcs, dsts], axis=1)
    zrow = jnp.zeros((NPT, 128), f32)
    ones = jnp.ones((CHUNK, 128), f32)
    xq = [x[:, 128 * q:128 * (q + 1)] for q in range(4)]
    bl2 = bl.reshape(1, 256)
    ba2 = ba.reshape(1, 128)
    b12 = b1.reshape(1, 64)
    b22 = b2.reshape(1, 32)
    w3t = jnp.pad(W3.T, ((0, 0), (0, 5)))
    b32 = jnp.pad(b3, (0, 5)).reshape(1, 8)

    a0, a1 = _sc_aggregate(xq[0], xq[1], idxs, zrow)
    a2, a3 = _sc_aggregate(xq[2], xq[3], idxs, zrow)
    cntl, cntr = _sc_count(idxs, zrow, ones)
    pmat, qtmat = _mlp_factors(a0, a1, a2, a3, cntl, cntr, x, Wl.T, bl2,
                               Wr.T, Wa.T, ba2, W1.T, b12, W2.T, b22,
                               w3t, b32)
    return _cdist(pmat, qtmat)


# speedup vs baseline: 1.0246x; 1.0246x over previous
import functools

import jax
import jax.numpy as jnp
from jax import lax
from jax.experimental import pallas as pl
from jax.experimental.pallas import tpu as pltpu
from jax.experimental.pallas import tpu_sc as plsc

N = 10000
E = 160000
NTILES = 16
NPT = 640
NPAD = NTILES * NPT
CHUNK = 128
GROUP = 8
NGROUP = 10
EPT_PAD = GROUP * NGROUP * CHUNK
E_PAD = EPT_PAD * NTILES


def _sc_aggregate(xa, xb, idxs, zrow):
    mesh = plsc.VectorSubcoreMesh(core_axis_name="c", subcore_axis_name="s")

    @functools.partial(
        pl.kernel,
        out_type=(
            jax.ShapeDtypeStruct((NPAD, 128), jnp.float32),
            jax.ShapeDtypeStruct((NPAD, 128), jnp.float32),
        ),
        mesh=mesh,
        scratch_types=[
            pltpu.VMEM_SHARED((NPAD, 128), jnp.float32),
            pltpu.VMEM((2 * GROUP, CHUNK), jnp.int32),
            pltpu.VMEM((CHUNK, 128), jnp.float32),
            pltpu.VMEM((CHUNK, 128), jnp.float32),
            pltpu.SemaphoreType.DMA,
            pltpu.SemaphoreType.DMA,
            pltpu.SemaphoreType.DMA,
            pltpu.SemaphoreType.DMA,
        ],
    )
    def k(xa_h, xb_h, idxs_h, zrow_h, agga_h, aggb_h,
          agg_s, idx_v, rowbuf0, rowbuf1, gsem0, gsem1, ssem0,
          ssem1):
        c = lax.axis_index("c")
        s = lax.axis_index("s")
        base = s * NPT
        sl = pl.ds(base, NPT)
        pltpu.sync_copy(zrow_h, agg_s.at[sl])
        plsc.subcore_barrier()

        bufs = (rowbuf0, rowbuf1)
        gsems = (gsem0, gsem1)
        ssems = (ssem0, ssem1)

        def run(x_h):
            def group(g, carry):
                pltpu.sync_copy(idxs_h.at[s * NGROUP + g], idx_v)
                gh = [None] * GROUP
                sh = [None] * GROUP
                gh[0] = pltpu.async_copy(x_h.at[idx_v.at[0]], bufs[0],
                                         gsems[0])
                for j in range(GROUP):
                    b = j % 2
                    gh[j].wait()
                    if j + 1 < GROUP:
                        if j >= 1:
                            sh[j - 1].wait()
                        gh[j + 1] = pltpu.async_copy(
                            x_h.at[idx_v.at[j + 1]], bufs[1 - b],
                            gsems[1 - b])
                    sh[j] = pltpu.async_copy(bufs[b],
                                             agg_s.at[idx_v.at[GROUP + j]],
                                             ssems[b], add=True)
                sh[GROUP - 2].wait()
                sh[GROUP - 1].wait()
                return carry
            lax.fori_loop(0, NGROUP, group, 0)

        @pl.when(c == 0)
        def _():
            run(xa_h)

        @pl.when(c == 1)
        def _():
            run(xb_h)

        plsc.subcore_barrier()

        @pl.when(c == 0)
        def _():
            pltpu.sync_copy(agg_s.at[sl], agga_h.at[sl])

        @pl.when(c == 1)
        def _():
            pltpu.sync_copy(agg_s.at[sl], aggb_h.at[sl])

    return k(xa, xb, idxs, zrow)


def _sc_count(idxs, zrow, ones):
    mesh = plsc.VectorSubcoreMesh(core_axis_name="c", subcore_axis_name="s")
    gpw = NTILES * NGROUP // 32

    @functools.partial(
        pl.kernel,
        out_type=(
            jax.ShapeDtypeStruct((NPAD, 128), jnp.float32),
            jax.ShapeDtypeStruct((NPAD, 128), jnp.float32),
        ),
        mesh=mesh,
        scratch_types=[
            pltpu.VMEM_SHARED((NPAD, 128), jnp.float32),
            pltpu.VMEM((2 * GROUP, CHUNK), jnp.int32),
            pltpu.VMEM((CHUNK, 128), jnp.float32),
            pltpu.SemaphoreType.DMA,
        ],
    )
    def k(idxs_h, zrow_h, ones_h, cntl_h, cntr_h, cnt_s, idx_v, ones_v,
          sem):
        c = lax.axis_index("c")
        s = lax.axis_index("s")
        base = s * NPT
        sl = pl.ds(base, NPT)
        gbase = (c * NTILES + s) * gpw
        pltpu.sync_copy(zrow_h, cnt_s.at[sl])
        pltpu.sync_copy(ones_h, ones_v)
        plsc.subcore_barrier()

        def group(g, carry):
            pltpu.sync_copy(idxs_h.at[gbase + g], idx_v)
            hs = [pltpu.async_copy(ones_v, cnt_s.at[idx_v.at[GROUP + j]],
                                   sem, add=True)
                  for j in range(GROUP)]
            for h in hs:
                h.wait()
            return carry
        lax.fori_loop(0, gpw, group, 0)

        plsc.subcore_barrier()

        @pl.when(c == 0)
        def _():
            pltpu.sync_copy(cnt_s.at[sl], cntl_h.at[sl])

        @pl.when(c == 1)
        def _():
            pltpu.sync_copy(cnt_s.at[sl], cntr_h.at[sl])

    return k(idxs, zrow, ones)


def _mlp_factors(a0, a1, a2, a3, cntl, cntr, x, wlt, bl2, wrt, wat, ba2,
                 w1t, b12, w2t, b22, w3t, b32):
    rb = 1024

    def body(r0, r1, r2, r3, cl, cr, xr, wl_r, bl_r, wr_r, wa_r, ba_r,
             w1_r, b1_r, w2_r, b2_r, w3_r, b3_r, p_ref, qt_ref):
        cntc = jnp.maximum(cl[:, :1] + cr[:, :1], 1.0)
        agg = jnp.concatenate([r0[...], r1[...], r2[...], r3[...]],
                              axis=1) / cntc
        h = (jnp.dot(agg, wl_r[...], preferred_element_type=jnp.float32)
             + bl_r[...]
             + jnp.dot(xr[...], wr_r[...],
                       preferred_element_type=jnp.float32))
        h = jnp.maximum(h, 0.0)
        h = jnp.maximum(
            jnp.dot(h, wa_r[...], preferred_element_type=jnp.float32)
            + ba_r[...], 0.0)
        h = jnp.maximum(
            jnp.dot(h, w1_r[...], preferred_element_type=jnp.float32)
            + b1_r[...], 0.0)
        h = jnp.maximum(
            jnp.dot(h, w2_r[...], preferred_element_type=jnp.float32)
            + b2_r[...], 0.0)
        z = (jnp.dot(h, w3_r[...], preferred_element_type=jnp.float32)
             + b3_r[...])
        sq = jnp.sum(z * z, axis=1, keepdims=True)
        col = lax.broadcasted_iota(jnp.int32, z.shape, 1)
        p = jnp.where(col < 3, z, jnp.where(col == 3, sq, 0.0))
        p_ref[...] = p
        qt_ref[...] = p.T

    full = lambda shape: pl.BlockSpec(shape, lambda i: (0, 0))
    return pl.pallas_call(
        body,
        grid=(pl.cdiv(N, rb),),
        in_specs=[
            pl.BlockSpec((rb, 128), lambda i: (i, 0)),
            pl.BlockSpec((rb, 128), lambda i: (i, 0)),
            pl.BlockSpec((rb, 128), lambda i: (i, 0)),
            pl.BlockSpec((rb, 128), lambda i: (i, 0)),
            pl.BlockSpec((rb, 128), lambda i: (i, 0)),
            pl.BlockSpec((rb, 128), lambda i: (i, 0)),
            pl.BlockSpec((rb, 512), lambda i: (i, 0)),
            full((512, 256)), full((1, 256)), full((512, 256)),
            full((256, 128)), full((1, 128)),
            full((128, 64)), full((1, 64)), full((64, 32)), full((1, 32)),
            full((32, 8)), full((1, 8)),
        ],
        out_specs=[
            pl.BlockSpec((rb, 8), lambda i: (i, 0)),
            pl.BlockSpec((8, rb), lambda i: (0, i)),
        ],
        out_shape=[
            jax.ShapeDtypeStruct((N, 8), jnp.float32),
            jax.ShapeDtypeStruct((8, N), jnp.float32),
        ],
    )(a0, a1, a2, a3, cntl, cntr, x, wlt, bl2, wrt, wat, ba2, w1t, b12,
      w2t, b22, w3t, b32)


def _cdist(p, qt):
    rb, cb = 512, 2048

    def body(p_ref, qt_ref, o_ref):
        pm = p_ref[...]
        qm = qt_ref[...]
        colp = lax.broadcasted_iota(jnp.int32, pm.shape, 1)
        colq = lax.broadcasted_iota(jnp.int32, qm.shape, 0)
        zi = jnp.where(colp < 3, pm, 0.0)
        zj = jnp.where(colq < 3, qm, 0.0)
        zz = jnp.dot(zi, zj, preferred_element_type=jnp.float32)
        d2 = (pm[:, 3:4] + qm[3:4, :]) - 2.0 * zz
        o_ref[...] = jnp.sqrt(jnp.maximum(d2, 1e-24))

    return pl.pallas_call(
        body,
        grid=(pl.cdiv(N, rb), pl.cdiv(N, cb)),
        in_specs=[
            pl.BlockSpec((rb, 8), lambda i, j: (i, 0)),
            pl.BlockSpec((8, cb), lambda i, j: (0, j)),
        ],
        out_specs=pl.BlockSpec((rb, cb), lambda i, j: (i, j)),
        out_shape=jax.ShapeDtypeStruct((N, N), jnp.float32),
    )(p, qt)


def kernel(x, edge_index, Wl, bl, Wr, Wa, ba, W1, b1, W2, b2, W3, b3):
    f32 = jnp.float32
    src = edge_index[0].astype(jnp.int32)
    dst = edge_index[1].astype(jnp.int32)
    npad_e = E_PAD - E
    pad_iota = jnp.arange(npad_e, dtype=jnp.int32)
    src_full = jnp.concatenate([src, pad_iota % N])
    dst_full = jnp.concatenate([dst, N + pad_iota % (NPAD - N)])
    srcs = src_full.reshape(NTILES * NGROUP, GROUP, CHUNK)
    dsts = dst_full.reshape(NTILES * NGROUP, GROUP, CHUNK)
    idxs = jnp.concatenate([srcs, dsts], axis=1)
    zrow = jnp.zeros((NPT, 128), f32)
    ones = jnp.ones((CHUNK, 128), f32)
    xq = [x[:, 128 * q:128 * (q + 1)] for q in range(4)]
    bl2 = bl.reshape(1, 256)
    ba2 = ba.reshape(1, 128)
    b12 = b1.reshape(1, 64)
    b22 = b2.reshape(1, 32)
    w3t = jnp.pad(W3.T, ((0, 0), (0, 5)))
    b32 = jnp.pad(b3, (0, 5)).reshape(1, 8)

    a0, a1 = _sc_aggregate(xq[0], xq[1], idxs, zrow)
    a2, a3 = _sc_aggregate(xq[2], xq[3], idxs, zrow)
    cntl, cntr = _sc_count(idxs, zrow, ones)
    pmat, qtmat = _mlp_factors(a0, a1, a2, a3, cntl, cntr, x, Wl.T, bl2,
                               Wr.T, Wa.T, ba2, W1.T, b12, W2.T, b22,
                               w3t, b32)
    return _cdist(pmat, qtmat)


# speedup vs baseline: 1.0700x; 1.0443x over previous
import functools

import jax
import jax.numpy as jnp
from jax import lax
from jax.experimental import pallas as pl
from jax.experimental.pallas import tpu as pltpu
from jax.experimental.pallas import tpu_sc as plsc

N = 10000
E = 160000
NTILES = 16
NPT = 640
NPAD = NTILES * NPT
CHUNK = 128
GROUP = 8
NGROUP = 10
EPT_PAD = GROUP * NGROUP * CHUNK
E_PAD = EPT_PAD * NTILES


def _sc_aggregate(xa, xb, idxs, zrow):
    mesh = plsc.VectorSubcoreMesh(core_axis_name="c", subcore_axis_name="s")

    @functools.partial(
        pl.kernel,
        out_type=(
            jax.ShapeDtypeStruct((NPAD, 128), jnp.float32),
            jax.ShapeDtypeStruct((NPAD, 128), jnp.float32),
        ),
        mesh=mesh,
        scratch_types=[
            pltpu.VMEM_SHARED((NPAD, 128), jnp.float32),
            pltpu.VMEM((2 * GROUP, CHUNK), jnp.int32),
            pltpu.VMEM((CHUNK, 128), jnp.float32),
            pltpu.VMEM((CHUNK, 128), jnp.float32),
            pltpu.SemaphoreType.DMA,
            pltpu.SemaphoreType.DMA,
            pltpu.SemaphoreType.DMA,
            pltpu.SemaphoreType.DMA,
        ],
    )
    def k(xa_h, xb_h, idxs_h, zrow_h, agga_h, aggb_h,
          agg_s, idx_v, rowbuf0, rowbuf1, gsem0, gsem1, ssem0,
          ssem1):
        c = lax.axis_index("c")
        s = lax.axis_index("s")
        base = s * NPT
        sl = pl.ds(base, NPT)
        pltpu.sync_copy(zrow_h, agg_s.at[sl])
        plsc.subcore_barrier()

        bufs = (rowbuf0, rowbuf1)
        gsems = (gsem0, gsem1)
        ssems = (ssem0, ssem1)

        def run(x_h):
            def group(g, carry):
                pltpu.sync_copy(idxs_h.at[s * NGROUP + g], idx_v)
                gh = [None] * GROUP
                sh = [None] * GROUP
                gh[0] = pltpu.async_copy(x_h.at[idx_v.at[0]], bufs[0],
                                         gsems[0])
                for j in range(GROUP):
                    b = j % 2
                    gh[j].wait()
                    if j + 1 < GROUP:
                        if j >= 1:
                            sh[j - 1].wait()
                        gh[j + 1] = pltpu.async_copy(
                            x_h.at[idx_v.at[j + 1]], bufs[1 - b],
                            gsems[1 - b])
                    sh[j] = pltpu.async_copy(bufs[b],
                                             agg_s.at[idx_v.at[GROUP + j]],
                                             ssems[b], add=True)
                sh[GROUP - 2].wait()
                sh[GROUP - 1].wait()
                return carry
            lax.fori_loop(0, NGROUP, group, 0)

        @pl.when(c == 0)
        def _():
            run(xa_h)

        @pl.when(c == 1)
        def _():
            run(xb_h)

        plsc.subcore_barrier()

        @pl.when(c == 0)
        def _():
            pltpu.sync_copy(agg_s.at[sl], agga_h.at[sl])

        @pl.when(c == 1)
        def _():
            pltpu.sync_copy(agg_s.at[sl], aggb_h.at[sl])

    return k(xa, xb, idxs, zrow)


def _sc_count(idxs, zrow, ones):
    mesh = plsc.VectorSubcoreMesh(core_axis_name="c", subcore_axis_name="s")
    gpw = NTILES * NGROUP // 32

    @functools.partial(
        pl.kernel,
        out_type=(
            jax.ShapeDtypeStruct((NPAD, 128), jnp.float32),
            jax.ShapeDtypeStruct((NPAD, 128), jnp.float32),
        ),
        mesh=mesh,
        scratch_types=[
            pltpu.VMEM_SHARED((NPAD, 128), jnp.float32),
            pltpu.VMEM((2 * GROUP, CHUNK), jnp.int32),
            pltpu.VMEM((CHUNK, 128), jnp.float32),
            pltpu.SemaphoreType.DMA,
        ],
    )
    def k(idxs_h, zrow_h, ones_h, cntl_h, cntr_h, cnt_s, idx_v, ones_v,
          sem):
        c = lax.axis_index("c")
        s = lax.axis_index("s")
        base = s * NPT
        sl = pl.ds(base, NPT)
        gbase = (c * NTILES + s) * gpw
        pltpu.sync_copy(zrow_h, cnt_s.at[sl])
        pltpu.sync_copy(ones_h, ones_v)
        plsc.subcore_barrier()

        def group(g, carry):
            pltpu.sync_copy(idxs_h.at[gbase + g], idx_v)
            hs = [pltpu.async_copy(ones_v, cnt_s.at[idx_v.at[GROUP + j]],
                                   sem, add=True)
                  for j in range(GROUP)]
            for h in hs:
                h.wait()
            return carry
        lax.fori_loop(0, gpw, group, 0)

        plsc.subcore_barrier()

        @pl.when(c == 0)
        def _():
            pltpu.sync_copy(cnt_s.at[sl], cntl_h.at[sl])

        @pl.when(c == 1)
        def _():
            pltpu.sync_copy(cnt_s.at[sl], cntr_h.at[sl])

    return k(idxs, zrow, ones)


def _mlp_factors(a0, a1, a2, a3, cntl, cntr, x, wlt, bl2, wrt, wat, ba2,
                 w1t, b12, w2t, b22, w3t, b32):
    rb = 1024

    def body(r0, r1, r2, r3, cl, cr, xr, wl_r, bl_r, wr_r, wa_r, ba_r,
             w1_r, b1_r, w2_r, b2_r, w3_r, b3_r, p_ref, qt_ref):
        cntc = jnp.maximum(cl[:, :1] + cr[:, :1], 1.0)
        agg = jnp.concatenate([r0[...], r1[...], r2[...], r3[...]],
                              axis=1) / cntc
        h = (jnp.dot(agg, wl_r[...], preferred_element_type=jnp.float32)
             + bl_r[...]
             + jnp.dot(xr[...], wr_r[...],
                       preferred_element_type=jnp.float32))
        h = jnp.maximum(h, 0.0)
        h = jnp.maximum(
            jnp.dot(h, wa_r[...], preferred_element_type=jnp.float32)
            + ba_r[...], 0.0)
        h = jnp.maximum(
            jnp.dot(h, w1_r[...], preferred_element_type=jnp.float32)
            + b1_r[...], 0.0)
        h = jnp.maximum(
            jnp.dot(h, w2_r[...], preferred_element_type=jnp.float32)
            + b2_r[...], 0.0)
        z = (jnp.dot(h, w3_r[...], preferred_element_type=jnp.float32)
             + b3_r[...])
        sq = jnp.sum(z * z, axis=1, keepdims=True)
        col = lax.broadcasted_iota(jnp.int32, z.shape, 1)
        p = jnp.where(col < 3, z, jnp.where(col == 3, sq, 0.0))
        p_ref[...] = p
        qt_ref[...] = p.T

    full = lambda shape: pl.BlockSpec(shape, lambda i: (0, 0))
    return pl.pallas_call(
        body,
        grid=(pl.cdiv(N, rb),),
        in_specs=[
            pl.BlockSpec((rb, 128), lambda i: (i, 0)),
            pl.BlockSpec((rb, 128), lambda i: (i, 0)),
            pl.BlockSpec((rb, 128), lambda i: (i, 0)),
            pl.BlockSpec((rb, 128), lambda i: (i, 0)),
            pl.BlockSpec((rb, 128), lambda i: (i, 0)),
            pl.BlockSpec((rb, 128), lambda i: (i, 0)),
            pl.BlockSpec((rb, 512), lambda i: (i, 0)),
            full((512, 256)), full((1, 256)), full((512, 256)),
            full((256, 128)), full((1, 128)),
            full((128, 64)), full((1, 64)), full((64, 32)), full((1, 32)),
            full((32, 8)), full((1, 8)),
        ],
        out_specs=[
            pl.BlockSpec((rb, 8), lambda i: (i, 0)),
            pl.BlockSpec((8, rb), lambda i: (0, i)),
        ],
        out_shape=[
            jax.ShapeDtypeStruct((N, 8), jnp.float32),
            jax.ShapeDtypeStruct((8, N), jnp.float32),
        ],
    )(a0, a1, a2, a3, cntl, cntr, x, wlt, bl2, wrt, wat, ba2, w1t, b12,
      w2t, b22, w3t, b32)


def _cdist(p, qt):
    rb, cb = 512, 2048

    def body(p_ref, qt_ref, o_ref):
        pm = p_ref[...]
        qm = qt_ref[...]
        colp = lax.broadcasted_iota(jnp.int32, pm.shape, 1)
        colq = lax.broadcasted_iota(jnp.int32, qm.shape, 0)
        zi = jnp.where(colp < 3, pm, 0.0)
        zj = jnp.where(colq < 3, -2.0 * qm, 0.0)
        zz = jnp.dot(zi, zj, preferred_element_type=jnp.float32)
        d2 = jnp.maximum((pm[:, 3:4] + qm[3:4, :]) + zz, 1e-24)
        o_ref[...] = d2 * lax.rsqrt(d2)

    return pl.pallas_call(
        body,
        grid=(pl.cdiv(N, rb), pl.cdiv(N, cb)),
        in_specs=[
            pl.BlockSpec((rb, 8), lambda i, j: (i, 0)),
            pl.BlockSpec((8, cb), lambda i, j: (0, j)),
        ],
        out_specs=pl.BlockSpec((rb, cb), lambda i, j: (i, j)),
        out_shape=jax.ShapeDtypeStruct((N, N), jnp.float32),
    )(p, qt)


def kernel(x, edge_index, Wl, bl, Wr, Wa, ba, W1, b1, W2, b2, W3, b3):
    f32 = jnp.float32
    src = edge_index[0].astype(jnp.int32)
    dst = edge_index[1].astype(jnp.int32)
    npad_e = E_PAD - E
    pad_iota = jnp.arange(npad_e, dtype=jnp.int32)
    src_full = jnp.concatenate([src, pad_iota % N])
    dst_full = jnp.concatenate([dst, N + pad_iota % (NPAD - N)])
    srcs = src_full.reshape(NTILES * NGROUP, GROUP, CHUNK)
    dsts = dst_full.reshape(NTILES * NGROUP, GROUP, CHUNK)
    idxs = jnp.concatenate([srcs, dsts], axis=1)
    zrow = jnp.zeros((NPT, 128), f32)
    ones = jnp.ones((CHUNK, 128), f32)
    xq = [x[:, 128 * q:128 * (q + 1)] for q in range(4)]
    bl2 = bl.reshape(1, 256)
    ba2 = ba.reshape(1, 128)
    b12 = b1.reshape(1, 64)
    b22 = b2.reshape(1, 32)
    w3t = jnp.pad(W3.T, ((0, 0), (0, 5)))
    b32 = jnp.pad(b3, (0, 5)).reshape(1, 8)

    a0, a1 = _sc_aggregate(xq[0], xq[1], idxs, zrow)
    a2, a3 = _sc_aggregate(xq[2], xq[3], idxs, zrow)
    cntl, cntr = _sc_count(idxs, zrow, ones)
    pmat, qtmat = _mlp_factors(a0, a1, a2, a3, cntl, cntr, x, Wl.T, bl2,
                               Wr.T, Wa.T, ba2, W1.T, b12, W2.T, b22,
                               w3t, b32)
    return _cdist(pmat, qtmat)
